# baseline (speedup 1.0000x reference)
import functools

import jax
import jax.numpy as jnp
from jax.experimental import pallas as pl
from jax.experimental.pallas import tpu as pltpu


def _tree_min(chunks):
    while len(chunks) > 1:
        nxt = [jnp.minimum(chunks[i], chunks[i + 1])
               for i in range(0, len(chunks) - 1, 2)]
        if len(chunks) % 2:
            nxt.append(chunks[-1])
        chunks = nxt
    return chunks[0]


def _chamfer_body(a_ref, bt_ref, s1_ref, s2_ref, rowacc_ref, *, n1, n2, tm):
    bt = bt_ref[0]

    def body(i, d2):
        atile = a_ref[0, pl.ds(i * tm, tm), :]
        pair = jnp.dot(atile, bt, preferred_element_type=jnp.float32)
        rowacc_ref[pl.ds(i * tm, tm), :] = _tree_min(
            [pair[:, k * 128:(k + 1) * 128] for k in range(n2 // 128)])
        cmin = _tree_min([pair[k * 8:(k + 1) * 8, :] for k in range(tm // 8)])
        return jnp.minimum(d2, cmin)

    d2 = jnp.full((8, n2), jnp.inf, dtype=jnp.float32)
    d2 = jax.lax.fori_loop(0, n1 // tm, body, d2)

    d1 = jnp.min(rowacc_ref[...], axis=1)
    s1 = jnp.sum(jnp.maximum(d1, 0.0))
    d2row = jnp.min(d2, axis=0)
    s2 = jnp.sum(jnp.maximum(d2row, 0.0))
    s1_ref[0] = jnp.full((1, 128), s1, dtype=jnp.float32)
    s2_ref[0] = jnp.full((1, 128), s2, dtype=jnp.float32)


def _split_hi_lo(sq):
    hi = sq.astype(jnp.bfloat16).astype(jnp.float32)
    return hi, sq - hi


def kernel(xyz1, xyz2):
    b, n1, _ = xyz1.shape
    _, n2, _ = xyz2.shape
    tm = 256

    sq1 = jnp.sum(xyz1 * xyz1, axis=-1, keepdims=True)
    sq2 = jnp.sum(xyz2 * xyz2, axis=-1, keepdims=True)
    s1a, s1b = _split_hi_lo(sq1)
    s2a, s2b = _split_hi_lo(sq2)
    one1 = jnp.ones_like(sq1)
    zero1 = jnp.zeros_like(sq1)
    one2 = jnp.ones_like(sq2)
    zero2 = jnp.zeros_like(sq2)
    aug1 = jnp.concatenate([xyz1, s1a, s1b, one1, one1, zero1], axis=-1)
    aug2 = jnp.concatenate([-2.0 * xyz2, one2, one2, s2a, s2b, zero2], axis=-1)
    aug2t = aug2.transpose(0, 2, 1)

    s1, s2 = pl.pallas_call(
        functools.partial(_chamfer_body, n1=n1, n2=n2, tm=tm),
        grid=(b,),
        in_specs=[
            pl.BlockSpec((1, n1, 8), lambda i: (i, 0, 0)),
            pl.BlockSpec((1, 8, n2), lambda i: (i, 0, 0)),
        ],
        out_specs=[
            pl.BlockSpec((1, 1, 128), lambda i: (i, 0, 0)),
            pl.BlockSpec((1, 1, 128), lambda i: (i, 0, 0)),
        ],
        out_shape=[
            jax.ShapeDtypeStruct((b, 1, 128), jnp.float32),
            jax.ShapeDtypeStruct((b, 1, 128), jnp.float32),
        ],
        scratch_shapes=[pltpu.VMEM((n1, 128), jnp.float32)],
        compiler_params=pltpu.CompilerParams(
            dimension_semantics=("parallel",),
        ),
    )(aug1, aug2t)

    return jnp.sum(s1[:, 0, 0]) / (b * n1) + jnp.sum(s2[:, 0, 0]) / (b * n2)


# speedup vs baseline: 1.1178x; 1.1178x over previous
import functools

import jax
import jax.numpy as jnp
from jax.experimental import pallas as pl
from jax.experimental.pallas import tpu as pltpu


def _chamfer_body(a_ref, bt_ref, s1_ref, s2_ref, *, n1, n2, tm):
    bt = bt_ref[0]

    def body(i, carry):
        s1, d2 = carry
        atile = a_ref[0, pl.ds(i * tm, tm), :]
        pair = jnp.dot(atile, bt, preferred_element_type=jnp.float32)
        rowmin = jnp.min(pair, axis=1)
        s1 = s1 + jnp.sum(jnp.maximum(rowmin, 0.0))
        d2 = jnp.minimum(d2, jnp.min(pair, axis=0, keepdims=True))
        return s1, d2

    s1 = jnp.float32(0.0)
    d2 = jnp.full((1, n2), jnp.inf, dtype=jnp.float32)
    s1, d2 = jax.lax.fori_loop(0, n1 // tm, body, (s1, d2), unroll=16)
    s2 = jnp.sum(jnp.maximum(d2, 0.0))
    s1_ref[0] = jnp.full((1, 128), s1, dtype=jnp.float32)
    s2_ref[0] = jnp.full((1, 128), s2, dtype=jnp.float32)


def _split_hi_lo(sq):
    hi = sq.astype(jnp.bfloat16).astype(jnp.float32)
    return hi, sq - hi


def kernel(xyz1, xyz2):
    b, n1, _ = xyz1.shape
    _, n2, _ = xyz2.shape
    tm = 256

    sq1 = jnp.sum(xyz1 * xyz1, axis=-1, keepdims=True)
    sq2 = jnp.sum(xyz2 * xyz2, axis=-1, keepdims=True)
    s1a, s1b = _split_hi_lo(sq1)
    s2a, s2b = _split_hi_lo(sq2)
    one1 = jnp.ones_like(sq1)
    zero1 = jnp.zeros_like(sq1)
    one2 = jnp.ones_like(sq2)
    zero2 = jnp.zeros_like(sq2)
    aug1 = jnp.concatenate([xyz1, s1a, s1b, one1, one1, zero1], axis=-1)
    aug2 = jnp.concatenate([-2.0 * xyz2, one2, one2, s2a, s2b, zero2], axis=-1)
    aug2t = aug2.transpose(0, 2, 1)

    s1, s2 = pl.pallas_call(
        functools.partial(_chamfer_body, n1=n1, n2=n2, tm=tm),
        grid=(b,),
        in_specs=[
            pl.BlockSpec((1, n1, 8), lambda i: (i, 0, 0)),
            pl.BlockSpec((1, 8, n2), lambda i: (i, 0, 0)),
        ],
        out_specs=[
            pl.BlockSpec((1, 1, 128), lambda i: (i, 0, 0)),
            pl.BlockSpec((1, 1, 128), lambda i: (i, 0, 0)),
        ],
        out_shape=[
            jax.ShapeDtypeStruct((b, 1, 128), jnp.float32),
            jax.ShapeDtypeStruct((b, 1, 128), jnp.float32),
        ],
        compiler_params=pltpu.CompilerParams(
            dimension_semantics=("parallel",),
        ),
    )(aug1, aug2t)

    return jnp.sum(s1[:, 0, 0]) / (b * n1) + jnp.sum(s2[:, 0, 0]) / (b * n2)
